# Initial kernel scaffold; baseline (speedup 1.0000x reference)
import jax
import jax.numpy as jnp
from jax.experimental import pallas as pl


def kernel(in_feat, adj_t, W1, b1, W2, b2):
    raise NotImplementedError("write your pallas kernel here")


# speedup vs baseline: 21.1283x; 21.1283x over previous
import functools

import jax
import jax.numpy as jnp
from jax import lax
from jax.experimental import pallas as pl
from jax.experimental.pallas import tpu as pltpu
from jax.experimental.pallas import tpu_sc as plsc

N = 10000
D = 128
DH = D // 2
NSC = 2
NSUB = 16
NW = NSC * NSUB
CH = 128
PAD = 240
NACC = N + PAD
ZROWS = NACC // NSUB
BN = 1000

_mesh = plsc.VectorSubcoreMesh(
    core_axis_name="c", subcore_axis_name="s", num_cores=NSC)

_f32 = jnp.float32

_sc_params = pltpu.CompilerParams(use_tc_tiling_on_sc=False)


def _make_deg_kernel(nch):
  @functools.partial(
      pl.kernel,
      out_type=jax.ShapeDtypeStruct((NSC, NACC, 16), _f32),
      mesh=_mesh,
      scratch_types=[
          pltpu.VMEM((nch, CH), jnp.int32),
          pltpu.VMEM((CH, 16), _f32),
          pltpu.VMEM((CH, 16), _f32),
          pltpu.VMEM_SHARED((NACC, 16), _f32),
      ],
      compiler_params=_sc_params,
  )
  def deg_kernel(dsts_hbm, out_hbm, dst_v, ones_v, zeros_v, deg_sh):
    c = lax.axis_index("c")
    s = lax.axis_index("s")
    wid = c * NSUB + s
    one = jnp.ones((16,), _f32)
    zero = jnp.zeros((16,), _f32)

    @pl.loop(0, CH)
    def _(r):
      ones_v[r, pl.ds(0, 16)] = one
      zeros_v[r, pl.ds(0, 16)] = zero

    base = s * ZROWS
    for j in range(ZROWS // CH):
      pltpu.sync_copy(zeros_v, deg_sh.at[pl.ds(base + j * CH, CH)])
    pltpu.sync_copy(dsts_hbm.at[wid], dst_v)
    plsc.subcore_barrier()

    @pl.loop(0, nch)
    def _(ch):
      pltpu.sync_copy(ones_v, deg_sh.at[dst_v.at[ch]], add=True)

    plsc.subcore_barrier()
    pltpu.sync_copy(deg_sh.at[pl.ds(base, ZROWS)],
                    out_hbm.at[c, pl.ds(base, ZROWS)])

  return deg_kernel


def _make_scatter_kernel(nch):
  @functools.partial(
      pl.kernel,
      out_type=jax.ShapeDtypeStruct((NSC, NACC, DH), _f32),
      mesh=_mesh,
      scratch_types=[
          pltpu.VMEM((nch, CH), jnp.int32),
          pltpu.VMEM((nch, CH), jnp.int32),
          pltpu.VMEM((CH, DH), _f32),
          pltpu.VMEM((CH, DH), _f32),
          pltpu.VMEM_SHARED((NACC, DH), _f32),
          pltpu.SemaphoreType.DMA,
          pltpu.SemaphoreType.DMA,
      ],
      compiler_params=_sc_params,
  )
  def scatter_kernel(xs_hbm, srcs_hbm, dsts_hbm, out_hbm,
                     src_v, dst_v, bufa, bufb, acc_sh, sema, semb):
    c = lax.axis_index("c")
    s = lax.axis_index("s")
    zero = jnp.zeros((16,), _f32)

    @pl.loop(0, CH)
    def _(r):
      for k in range(DH // 16):
        bufa[r, pl.ds(k * 16, 16)] = zero

    base = s * ZROWS
    for j in range(ZROWS // CH):
      pltpu.sync_copy(bufa, acc_sh.at[pl.ds(base + j * CH, CH)])
    pltpu.sync_copy(srcs_hbm.at[c, s], src_v)
    pltpu.sync_copy(dsts_hbm.at[s], dst_v)
    plsc.subcore_barrier()

    st = 1 if nch % 2 else 2
    for ch0 in range(st):
      pltpu.async_copy(xs_hbm.at[src_v.at[ch0]], bufa, sema).wait()
      pltpu.sync_copy(bufa, acc_sh.at[dst_v.at[ch0]], add=True)
    pltpu.async_copy(xs_hbm.at[src_v.at[st]], bufb, semb)

    @pl.loop(st, nch, step=2)
    def _(ch):
      pltpu.async_copy(xs_hbm.at[src_v.at[ch + 1]], bufa, sema)
      pltpu.make_async_copy(xs_hbm.at[src_v.at[ch]], bufb, semb).wait()
      pltpu.sync_copy(bufb, acc_sh.at[dst_v.at[ch]], add=True)
      pltpu.make_async_copy(xs_hbm.at[src_v.at[ch + 1]], bufa, sema).wait()
      pltpu.sync_copy(bufa, acc_sh.at[dst_v.at[ch + 1]], add=True)

      @pl.when(ch + 2 < nch)
      def _():
        pltpu.async_copy(xs_hbm.at[src_v.at[ch + 2]], bufb, semb)

    plsc.subcore_barrier()
    pltpu.sync_copy(acc_sh.at[pl.ds(base, ZROWS)],
                    out_hbm.at[c, pl.ds(base, ZROWS)])

  return scatter_kernel


def _dot(a, b):
  return lax.dot_general(a, b, (((1,), (0,)), ((), ())),
                         precision=lax.Precision.HIGHEST,
                         preferred_element_type=_f32)


def _mm_body(x_ref, w_ref, o_ref):
  o_ref[...] = _dot(x_ref[...], w_ref[...])


def _tc_matmul(x, w):
  n = x.shape[0]
  return pl.pallas_call(
      _mm_body,
      grid=(n // BN,),
      in_specs=[pl.BlockSpec((BN, D), lambda i: (i, 0)),
                pl.BlockSpec((D, D), lambda i: (0, 0))],
      out_specs=pl.BlockSpec((BN, D), lambda i: (i, 0)),
      out_shape=jax.ShapeDtypeStruct((n, D), _f32),
  )(x, w)


def _scale_body(xw_ref, degp_ref, xs_ref, dinv_ref):
  d = degp_ref[0, :, 0:1] + degp_ref[1, :, 0:1]
  dinv = jnp.where(d > 0, lax.rsqrt(jnp.maximum(d, 1e-12)),
                   jnp.zeros_like(d))
  xs = xw_ref[...] * dinv
  xs_ref[0] = xs[:, :DH]
  xs_ref[1] = xs[:, DH:]
  dinv_ref[...] = dinv


def _tc_scale(xw, degp):
  return pl.pallas_call(
      _scale_body,
      grid=(N // BN,),
      in_specs=[pl.BlockSpec((BN, D), lambda i: (i, 0)),
                pl.BlockSpec((NSC, BN, 16), lambda i: (0, i, 0))],
      out_specs=[pl.BlockSpec((NSC, BN, DH), lambda i: (0, i, 0)),
                 pl.BlockSpec((BN, 1), lambda i: (i, 0))],
      out_shape=[jax.ShapeDtypeStruct((NSC, N, DH), _f32),
                 jax.ShapeDtypeStruct((N, 1), _f32)],
  )(xw, degp)


def _mid_body(y_ref, dinv_ref, b_ref, w_ref, o_ref):
  dinv = dinv_ref[...]
  h = jnp.concatenate([y_ref[0], y_ref[1]], axis=1) * dinv + b_ref[...]
  xs = _dot(h, w_ref[...]) * dinv
  o_ref[0] = xs[:, :DH]
  o_ref[1] = xs[:, DH:]


def _tc_mid(y, dinv, b, w):
  return pl.pallas_call(
      _mid_body,
      grid=(N // BN,),
      in_specs=[pl.BlockSpec((NSC, BN, DH), lambda i: (0, i, 0)),
                pl.BlockSpec((BN, 1), lambda i: (i, 0)),
                pl.BlockSpec((1, D), lambda i: (0, 0)),
                pl.BlockSpec((D, D), lambda i: (0, 0))],
      out_specs=pl.BlockSpec((NSC, BN, DH), lambda i: (0, i, 0)),
      out_shape=jax.ShapeDtypeStruct((NSC, N, DH), _f32),
  )(y, dinv, b, w)


def _fin_body(y_ref, dinv_ref, b_ref, o_ref):
  o_ref[...] = (jnp.concatenate([y_ref[0], y_ref[1]], axis=1)
                * dinv_ref[...] + b_ref[...])


def _tc_fin(y, dinv, b):
  return pl.pallas_call(
      _fin_body,
      grid=(N // BN,),
      in_specs=[pl.BlockSpec((NSC, BN, DH), lambda i: (0, i, 0)),
                pl.BlockSpec((BN, 1), lambda i: (i, 0)),
                pl.BlockSpec((1, D), lambda i: (0, 0))],
      out_specs=pl.BlockSpec((BN, D), lambda i: (i, 0)),
      out_shape=jax.ShapeDtypeStruct((N, D), _f32),
  )(y, dinv, b)


def kernel(in_feat, adj_t, W1, b1, W2, b2):
  e = adj_t.shape[1]
  e_tot = e + N
  nch32 = -(-e_tot // (NW * CH))
  e_pad = NW * nch32 * CH
  nch16 = NSC * nch32
  npad = e_pad - e_tot

  loop_idx = jnp.arange(N, dtype=jnp.int32)
  ar = jnp.arange(npad, dtype=jnp.int32)
  src_p = jnp.concatenate([adj_t[0], loop_idx, ar % 16])
  dst_p = jnp.concatenate([adj_t[1], loop_idx, N + (ar % PAD)])
  dsts32 = dst_p.reshape(NW, nch32, CH)
  srcs16 = src_p.reshape(NSUB, nch16, CH)
  srcs2 = jnp.stack([srcs16, srcs16 + N])
  dsts16 = dst_p.reshape(NSUB, nch16, CH)

  deg_k = _make_deg_kernel(nch32)
  scat_k = _make_scatter_kernel(nch16)

  degp = deg_k(dsts32)
  xw1 = _tc_matmul(in_feat, W1)
  xs1, dinv = _tc_scale(xw1, degp)
  y1 = scat_k(xs1.reshape(NSC * N, DH), srcs2, dsts16)
  xs2 = _tc_mid(y1, dinv, b1.reshape(1, D), W2)
  y2 = scat_k(xs2.reshape(NSC * N, DH), srcs2, dsts16)
  return _tc_fin(y2, dinv, b2.reshape(1, D))


# speedup vs baseline: 25.7637x; 1.2194x over previous
import functools

import jax
import jax.numpy as jnp
from jax import lax
from jax.experimental import pallas as pl
from jax.experimental.pallas import tpu as pltpu
from jax.experimental.pallas import tpu_sc as plsc

N = 10000
D = 128
DH = D // 2
NSC = 2
NSUB = 16
NW = NSC * NSUB
CH = 128
PAD = 240
NACC = N + PAD
ZROWS = NACC // NSUB
BN = 1000

_mesh = plsc.VectorSubcoreMesh(
    core_axis_name="c", subcore_axis_name="s", num_cores=NSC)

_f32 = jnp.float32

_sc_params = pltpu.CompilerParams(use_tc_tiling_on_sc=False)


def _make_deg_kernel(nch):
  @functools.partial(
      pl.kernel,
      out_type=jax.ShapeDtypeStruct((NSC, NACC, 16), _f32),
      mesh=_mesh,
      scratch_types=[
          pltpu.VMEM((nch, CH), jnp.int32),
          pltpu.VMEM((CH, 16), _f32),
          pltpu.VMEM((CH, 16), _f32),
          pltpu.VMEM_SHARED((NACC, 16), _f32),
      ],
      compiler_params=_sc_params,
  )
  def deg_kernel(dsts_hbm, out_hbm, dst_v, ones_v, zeros_v, deg_sh):
    c = lax.axis_index("c")
    s = lax.axis_index("s")
    wid = c * NSUB + s
    one = jnp.ones((16,), _f32)
    zero = jnp.zeros((16,), _f32)

    @pl.loop(0, CH)
    def _(r):
      ones_v[r, pl.ds(0, 16)] = one
      zeros_v[r, pl.ds(0, 16)] = zero

    base = s * ZROWS
    for j in range(ZROWS // CH):
      pltpu.sync_copy(zeros_v, deg_sh.at[pl.ds(base + j * CH, CH)])
    pltpu.sync_copy(dsts_hbm.at[wid], dst_v)
    plsc.subcore_barrier()

    @pl.loop(0, nch)
    def _(ch):
      pltpu.sync_copy(ones_v, deg_sh.at[dst_v.at[ch]], add=True)

    plsc.subcore_barrier()
    pltpu.sync_copy(deg_sh.at[pl.ds(base, ZROWS)],
                    out_hbm.at[c, pl.ds(base, ZROWS)])

  return deg_kernel


def _make_scatter_kernel(nch):
  G = 2
  ngrp = (nch // (2 * G)) * 2
  pipelined = ngrp >= 4

  @functools.partial(
      pl.kernel,
      out_type=jax.ShapeDtypeStruct((NSC, NACC, DH), _f32),
      mesh=_mesh,
      scratch_types=[
          pltpu.VMEM((nch, CH), jnp.int32),
          pltpu.VMEM((nch, CH), jnp.int32),
      ] + [pltpu.VMEM((CH, DH), _f32) for _ in range(2 * G)] + [
          pltpu.VMEM_SHARED((NACC, DH), _f32),
          pltpu.SemaphoreType.DMA,
          pltpu.SemaphoreType.DMA,
          pltpu.SemaphoreType.DMA,
          pltpu.SemaphoreType.DMA,
      ],
      compiler_params=_sc_params,
  )
  def scatter_kernel(xs_hbm, srcs_hbm, dsts_hbm, out_hbm,
                     src_v, dst_v, *rest):
    bufs = rest[:2 * G]
    seta, setb = bufs[:G], bufs[G:]
    acc_sh, gsa, gsb, ssa, ssb = rest[2 * G:]
    c = lax.axis_index("c")
    s = lax.axis_index("s")
    zero = jnp.zeros((16,), _f32)
    buf0 = seta[0]

    @pl.loop(0, CH)
    def _(r):
      for k in range(DH // 16):
        buf0[r, pl.ds(k * 16, 16)] = zero

    base = s * ZROWS
    for j in range(ZROWS // CH):
      pltpu.sync_copy(buf0, acc_sh.at[pl.ds(base + j * CH, CH)])
    pltpu.sync_copy(srcs_hbm.at[c, s], src_v)
    pltpu.sync_copy(dsts_hbm.at[s], dst_v)
    plsc.subcore_barrier()

    def fire_g(cb, st, sem):
      for j in range(G):
        pltpu.async_copy(xs_hbm.at[src_v.at[cb + j]], st[j], sem)

    def wait_g(st, sem):
      for j in range(G):
        pltpu.make_async_copy(xs_hbm.at[src_v.at[0]], st[j], sem).wait()

    def fire_s(cb, st, sem):
      for j in range(G):
        pltpu.async_copy(st[j], acc_sh.at[dst_v.at[cb + j]], sem, add=True)

    def wait_s(st, sem):
      for j in range(G):
        pltpu.make_async_copy(st[j], acc_sh.at[dst_v.at[0]], sem).wait()

    if pipelined:
      fire_g(0, seta, gsa)
      fire_g(G, setb, gsb)
      wait_g(seta, gsa)
      fire_s(0, seta, ssa)
      wait_g(setb, gsb)
      fire_s(G, setb, ssb)
      wait_s(seta, ssa)
      fire_g(2 * G, seta, gsa)

      @pl.loop(2 * G, ngrp * G, step=2 * G)
      def _(cb):
        wait_g(seta, gsa)
        fire_s(cb, seta, ssa)
        wait_s(setb, ssb)
        fire_g(cb + G, setb, gsb)
        wait_g(setb, gsb)
        fire_s(cb + G, setb, ssb)
        wait_s(seta, ssa)

        @pl.when(cb + 2 * G < ngrp * G)
        def _():
          fire_g(cb + 2 * G, seta, gsa)

      wait_s(setb, ssb)
      done = ngrp * G
    else:
      done = 0

    for ch0 in range(done, nch):
      st = seta[ch0 % G]
      pltpu.async_copy(xs_hbm.at[src_v.at[ch0]], st, gsa).wait()
      pltpu.sync_copy(st, acc_sh.at[dst_v.at[ch0]], add=True)

    plsc.subcore_barrier()
    pltpu.sync_copy(acc_sh.at[pl.ds(base, ZROWS)],
                    out_hbm.at[c, pl.ds(base, ZROWS)])

  return scatter_kernel


def _dot(a, b):
  return lax.dot_general(a, b, (((1,), (0,)), ((), ())),
                         precision=lax.Precision.HIGHEST,
                         preferred_element_type=_f32)


def _mm_scale_body(x_ref, w_ref, degp_ref, xs_ref, dinv_ref):
  d = degp_ref[0, :, 0:1] + degp_ref[1, :, 0:1]
  dinv = jnp.where(d > 0, lax.rsqrt(jnp.maximum(d, 1e-12)),
                   jnp.zeros_like(d))
  xs = _dot(x_ref[...], w_ref[...]) * dinv
  xs_ref[0] = xs[:, :DH]
  xs_ref[1] = xs[:, DH:]
  dinv_ref[...] = dinv


def _tc_mm_scale(x, w, degp):
  return pl.pallas_call(
      _mm_scale_body,
      grid=(N // BN,),
      in_specs=[pl.BlockSpec((BN, D), lambda i: (i, 0)),
                pl.BlockSpec((D, D), lambda i: (0, 0)),
                pl.BlockSpec((NSC, BN, 16), lambda i: (0, i, 0))],
      out_specs=[pl.BlockSpec((NSC, BN, DH), lambda i: (0, i, 0)),
                 pl.BlockSpec((BN, 1), lambda i: (i, 0))],
      out_shape=[jax.ShapeDtypeStruct((NSC, N, DH), _f32),
                 jax.ShapeDtypeStruct((N, 1), _f32)],
  )(x, w, degp)


def _mid_body(y_ref, dinv_ref, b_ref, w_ref, o_ref):
  dinv = dinv_ref[...]
  h = jnp.concatenate([y_ref[0], y_ref[1]], axis=1) * dinv + b_ref[...]
  xs = _dot(h, w_ref[...]) * dinv
  o_ref[0] = xs[:, :DH]
  o_ref[1] = xs[:, DH:]


def _tc_mid(y, dinv, b, w):
  return pl.pallas_call(
      _mid_body,
      grid=(N // BN,),
      in_specs=[pl.BlockSpec((NSC, BN, DH), lambda i: (0, i, 0)),
                pl.BlockSpec((BN, 1), lambda i: (i, 0)),
                pl.BlockSpec((1, D), lambda i: (0, 0)),
                pl.BlockSpec((D, D), lambda i: (0, 0))],
      out_specs=pl.BlockSpec((NSC, BN, DH), lambda i: (0, i, 0)),
      out_shape=jax.ShapeDtypeStruct((NSC, N, DH), _f32),
  )(y, dinv, b, w)


def _fin_body(y_ref, dinv_ref, b_ref, o_ref):
  o_ref[...] = (jnp.concatenate([y_ref[0], y_ref[1]], axis=1)
                * dinv_ref[...] + b_ref[...])


def _tc_fin(y, dinv, b):
  return pl.pallas_call(
      _fin_body,
      grid=(N // BN,),
      in_specs=[pl.BlockSpec((NSC, BN, DH), lambda i: (0, i, 0)),
                pl.BlockSpec((BN, 1), lambda i: (i, 0)),
                pl.BlockSpec((1, D), lambda i: (0, 0))],
      out_specs=pl.BlockSpec((BN, D), lambda i: (i, 0)),
      out_shape=jax.ShapeDtypeStruct((N, D), _f32),
  )(y, dinv, b)


def kernel(in_feat, adj_t, W1, b1, W2, b2):
  e = adj_t.shape[1]
  e_tot = e + N
  nch32 = -(-e_tot // (NW * CH))
  e_pad = NW * nch32 * CH
  nch16 = NSC * nch32
  npad = e_pad - e_tot

  loop_idx = jnp.arange(N, dtype=jnp.int32)
  ar = jnp.arange(npad, dtype=jnp.int32)
  src_p = jnp.concatenate([adj_t[0], loop_idx, ar % 16])
  dst_p = jnp.concatenate([adj_t[1], loop_idx, N + (ar % PAD)])
  dsts32 = dst_p.reshape(NW, nch32, CH)
  srcs16 = src_p.reshape(NSUB, nch16, CH)
  srcs2 = jnp.stack([srcs16, srcs16 + N])
  dsts16 = dst_p.reshape(NSUB, nch16, CH)

  deg_k = _make_deg_kernel(nch32)
  scat_k = _make_scatter_kernel(nch16)

  degp = deg_k(dsts32)
  xs1, dinv = _tc_mm_scale(in_feat, W1, degp)
  y1 = scat_k(xs1.reshape(NSC * N, DH), srcs2, dsts16)
  xs2 = _tc_mid(y1, dinv, b1.reshape(1, D), W2)
  y2 = scat_k(xs2.reshape(NSC * N, DH), srcs2, dsts16)
  return _tc_fin(y2, dinv, b2.reshape(1, D))


# speedup vs baseline: 26.0097x; 1.0095x over previous
import functools

import jax
import jax.numpy as jnp
from jax import lax
from jax.experimental import pallas as pl
from jax.experimental.pallas import tpu as pltpu
from jax.experimental.pallas import tpu_sc as plsc

N = 10000
D = 128
DH = D // 2
NSC = 2
NSUB = 16
NW = NSC * NSUB
CH = 128
PAD = 240
NACC = N + PAD
ZROWS = NACC // NSUB
BN = 1000

_mesh = plsc.VectorSubcoreMesh(
    core_axis_name="c", subcore_axis_name="s", num_cores=NSC)

_f32 = jnp.float32

_sc_params = pltpu.CompilerParams(use_tc_tiling_on_sc=False)


def _make_deg_kernel(nch):
  @functools.partial(
      pl.kernel,
      out_type=jax.ShapeDtypeStruct((NSC, NACC, 16), _f32),
      mesh=_mesh,
      scratch_types=[
          pltpu.VMEM((nch, CH), jnp.int32),
          pltpu.VMEM((CH, 16), _f32),
          pltpu.VMEM((CH, 16), _f32),
          pltpu.VMEM_SHARED((NACC, 16), _f32),
      ],
      compiler_params=_sc_params,
  )
  def deg_kernel(dsts_hbm, out_hbm, dst_v, ones_v, zeros_v, deg_sh):
    c = lax.axis_index("c")
    s = lax.axis_index("s")
    wid = c * NSUB + s
    one = jnp.ones((16,), _f32)
    zero = jnp.zeros((16,), _f32)

    @pl.loop(0, CH)
    def _(r):
      ones_v[r, pl.ds(0, 16)] = one
      zeros_v[r, pl.ds(0, 16)] = zero

    base = s * ZROWS
    for j in range(ZROWS // CH):
      pltpu.sync_copy(zeros_v, deg_sh.at[pl.ds(base + j * CH, CH)])
    pltpu.sync_copy(dsts_hbm.at[wid], dst_v)
    plsc.subcore_barrier()

    @pl.loop(0, nch)
    def _(ch):
      pltpu.sync_copy(ones_v, deg_sh.at[dst_v.at[ch]], add=True)

    plsc.subcore_barrier()
    pltpu.sync_copy(deg_sh.at[pl.ds(base, ZROWS)],
                    out_hbm.at[c, pl.ds(base, ZROWS)])

  return deg_kernel


def _make_scatter_kernel(nch):
  G = 4
  PH = 48
  phases = [min(PH, nch - p * PH) for p in range(-(-nch // PH))]

  @functools.partial(
      pl.kernel,
      out_type=jax.ShapeDtypeStruct((NSC, NACC, DH), _f32),
      mesh=_mesh,
      scratch_types=[
          pltpu.VMEM((PH, CH), jnp.int32),
          pltpu.VMEM((PH, CH), jnp.int32),
      ] + [pltpu.VMEM((CH, DH), _f32) for _ in range(2 * G)] + [
          pltpu.VMEM_SHARED((NACC, DH), _f32),
          pltpu.SemaphoreType.DMA,
          pltpu.SemaphoreType.DMA,
          pltpu.SemaphoreType.DMA,
          pltpu.SemaphoreType.DMA,
      ],
      compiler_params=_sc_params,
  )
  def scatter_kernel(xs_hbm, srcs_hbm, dsts_hbm, out_hbm,
                     src_v, dst_v, *rest):
    bufs = rest[:2 * G]
    seta, setb = bufs[:G], bufs[G:]
    acc_sh, gsa, gsb, ssa, ssb = rest[2 * G:]
    c = lax.axis_index("c")
    s = lax.axis_index("s")
    zero = jnp.zeros((16,), _f32)
    buf0 = seta[0]

    @pl.loop(0, CH)
    def _(r):
      for k in range(DH // 16):
        buf0[r, pl.ds(k * 16, 16)] = zero

    base = s * ZROWS
    for j in range(ZROWS // CH):
      pltpu.sync_copy(buf0, acc_sh.at[pl.ds(base + j * CH, CH)])
    plsc.subcore_barrier()

    def fire_g(cb, st, sem):
      for j in range(G):
        pltpu.async_copy(xs_hbm.at[src_v.at[cb + j]], st[j], sem)

    def wait_g(st, sem):
      for j in range(G):
        pltpu.make_async_copy(xs_hbm.at[src_v.at[0]], st[j], sem).wait()

    def fire_s(cb, st, sem):
      for j in range(G):
        pltpu.async_copy(st[j], acc_sh.at[dst_v.at[cb + j]], sem, add=True)

    def wait_s(st, sem):
      for j in range(G):
        pltpu.make_async_copy(st[j], acc_sh.at[dst_v.at[0]], sem).wait()

    for p, plen in enumerate(phases):
      pltpu.sync_copy(srcs_hbm.at[c, s, pl.ds(p * PH, plen)],
                      src_v.at[pl.ds(0, plen)])
      pltpu.sync_copy(dsts_hbm.at[s, pl.ds(p * PH, plen)],
                      dst_v.at[pl.ds(0, plen)])

      ngrp = (plen // (2 * G)) * 2
      if ngrp >= 4:
        fire_g(0, seta, gsa)
        fire_g(G, setb, gsb)
        wait_g(seta, gsa)
        fire_s(0, seta, ssa)
        wait_g(setb, gsb)
        fire_s(G, setb, ssb)
        wait_s(seta, ssa)
        fire_g(2 * G, seta, gsa)

        @pl.loop(2 * G, ngrp * G, step=2 * G)
        def _(cb):
          wait_g(seta, gsa)
          fire_s(cb, seta, ssa)
          wait_s(setb, ssb)
          fire_g(cb + G, setb, gsb)
          wait_g(setb, gsb)
          fire_s(cb + G, setb, ssb)
          wait_s(seta, ssa)

          @pl.when(cb + 2 * G < ngrp * G)
          def _():
            fire_g(cb + 2 * G, seta, gsa)

        wait_s(setb, ssb)
        done = ngrp * G
      else:
        done = 0

      for ch0 in range(done, plen):
        st = seta[ch0 % G]
        pltpu.async_copy(xs_hbm.at[src_v.at[ch0]], st, gsa).wait()
        pltpu.sync_copy(st, acc_sh.at[dst_v.at[ch0]], add=True)

    plsc.subcore_barrier()
    pltpu.sync_copy(acc_sh.at[pl.ds(base, ZROWS)],
                    out_hbm.at[c, pl.ds(base, ZROWS)])

  return scatter_kernel


def _dot(a, b):
  return lax.dot_general(a, b, (((1,), (0,)), ((), ())),
                         precision=lax.Precision.HIGHEST,
                         preferred_element_type=_f32)


def _mm_scale_body(x_ref, w_ref, degp_ref, xs_ref, dinv_ref):
  d = degp_ref[0, :, 0:1] + degp_ref[1, :, 0:1]
  dinv = jnp.where(d > 0, lax.rsqrt(jnp.maximum(d, 1e-12)),
                   jnp.zeros_like(d))
  xs = _dot(x_ref[...], w_ref[...]) * dinv
  xs_ref[0] = xs[:, :DH]
  xs_ref[1] = xs[:, DH:]
  dinv_ref[...] = dinv


def _tc_mm_scale(x, w, degp):
  return pl.pallas_call(
      _mm_scale_body,
      grid=(N // BN,),
      in_specs=[pl.BlockSpec((BN, D), lambda i: (i, 0)),
                pl.BlockSpec((D, D), lambda i: (0, 0)),
                pl.BlockSpec((NSC, BN, 16), lambda i: (0, i, 0))],
      out_specs=[pl.BlockSpec((NSC, BN, DH), lambda i: (0, i, 0)),
                 pl.BlockSpec((BN, 1), lambda i: (i, 0))],
      out_shape=[jax.ShapeDtypeStruct((NSC, N, DH), _f32),
                 jax.ShapeDtypeStruct((N, 1), _f32)],
  )(x, w, degp)


def _mid_body(y_ref, dinv_ref, b_ref, w_ref, o_ref):
  dinv = dinv_ref[...]
  h = jnp.concatenate([y_ref[0], y_ref[1]], axis=1) * dinv + b_ref[...]
  xs = _dot(h, w_ref[...]) * dinv
  o_ref[0] = xs[:, :DH]
  o_ref[1] = xs[:, DH:]


def _tc_mid(y, dinv, b, w):
  return pl.pallas_call(
      _mid_body,
      grid=(N // BN,),
      in_specs=[pl.BlockSpec((NSC, BN, DH), lambda i: (0, i, 0)),
                pl.BlockSpec((BN, 1), lambda i: (i, 0)),
                pl.BlockSpec((1, D), lambda i: (0, 0)),
                pl.BlockSpec((D, D), lambda i: (0, 0))],
      out_specs=pl.BlockSpec((NSC, BN, DH), lambda i: (0, i, 0)),
      out_shape=jax.ShapeDtypeStruct((NSC, N, DH), _f32),
  )(y, dinv, b, w)


def _fin_body(y_ref, dinv_ref, b_ref, o_ref):
  o_ref[...] = (jnp.concatenate([y_ref[0], y_ref[1]], axis=1)
                * dinv_ref[...] + b_ref[...])


def _tc_fin(y, dinv, b):
  return pl.pallas_call(
      _fin_body,
      grid=(N // BN,),
      in_specs=[pl.BlockSpec((NSC, BN, DH), lambda i: (0, i, 0)),
                pl.BlockSpec((BN, 1), lambda i: (i, 0)),
                pl.BlockSpec((1, D), lambda i: (0, 0))],
      out_specs=pl.BlockSpec((BN, D), lambda i: (i, 0)),
      out_shape=jax.ShapeDtypeStruct((N, D), _f32),
  )(y, dinv, b)


def kernel(in_feat, adj_t, W1, b1, W2, b2):
  e = adj_t.shape[1]
  e_tot = e + N
  nch32 = -(-e_tot // (NW * CH))
  e_pad = NW * nch32 * CH
  nch16 = NSC * nch32
  npad = e_pad - e_tot

  loop_idx = jnp.arange(N, dtype=jnp.int32)
  ar = jnp.arange(npad, dtype=jnp.int32)
  src_p = jnp.concatenate([adj_t[0], loop_idx, ar % 16])
  dst_p = jnp.concatenate([adj_t[1], loop_idx, N + (ar % PAD)])
  dsts32 = dst_p.reshape(NW, nch32, CH)
  srcs16 = src_p.reshape(NSUB, nch16, CH)
  srcs2 = jnp.stack([srcs16, srcs16 + N])
  dsts16 = dst_p.reshape(NSUB, nch16, CH)

  deg_k = _make_deg_kernel(nch32)
  scat_k = _make_scatter_kernel(nch16)

  degp = deg_k(dsts32)
  xs1, dinv = _tc_mm_scale(in_feat, W1, degp)
  y1 = scat_k(xs1.reshape(NSC * N, DH), srcs2, dsts16)
  xs2 = _tc_mid(y1, dinv, b1.reshape(1, D), W2)
  y2 = scat_k(xs2.reshape(NSC * N, DH), srcs2, dsts16)
  return _tc_fin(y2, dinv, b2.reshape(1, D))
